# baseline (speedup 1.0000x reference)
import functools

import jax
import jax.numpy as jnp
from jax import lax
from jax.experimental import pallas as pl
from jax.experimental.pallas import tpu as pltpu
from jax.experimental.pallas import tpu_sc as plsc

D = 768
E = 16
F = 2048
T = 2048
M = 128
MF = 256
NT = T // M
EP = 128

NC = 2
NS = 16
NW = NC * NS
RPW = T // NW


def _gate_body(x_ref, wg_ref, bg_ref, pos_ref, off_ref, hi_ref):
    xx = x_ref[...]
    logits16 = jnp.dot(xx, wg_ref[...],
                       preferred_element_type=jnp.float32) + bg_ref[...]
    logits = jnp.concatenate(
        [logits16, jnp.full((T, EP - E), -1e30, jnp.float32)], axis=1)
    lane = lax.broadcasted_iota(jnp.int32, (T, EP), 1)
    mx = jnp.max(logits, axis=-1, keepdims=True)
    e_t = jnp.min(jnp.where(logits == mx, lane, EP), axis=-1, keepdims=True)
    oh = (lane == e_t).astype(jnp.float32)

    counts = jnp.sum(oh, axis=0, keepdims=True)

    r = lax.broadcasted_iota(jnp.int32, (EP, EP), 0)
    c = lax.broadcasted_iota(jnp.int32, (EP, EP), 1)
    lt = (r < c).astype(jnp.float32)
    off = jnp.dot(counts, lt, preferred_element_type=jnp.float32)
    off_ref[...] = off.astype(jnp.int32)
    hi_ref[...] = (off + counts).astype(jnp.int32)

    ri = lax.broadcasted_iota(jnp.int32, (M, M), 0)
    ci = lax.broadcasted_iota(jnp.int32, (M, M), 1)
    tri = (ci <= ri).astype(jnp.float32)
    base = jnp.zeros((1, EP), jnp.float32)
    for b in range(NT):
        ohb = oh[b * M:(b + 1) * M]
        incb = jnp.dot(tri, ohb, preferred_element_type=jnp.float32) + base
        rank = jnp.sum(incb * ohb, axis=-1, keepdims=True) - 1.0
        offt = jnp.sum(off * ohb, axis=-1, keepdims=True)
        pos_ref[b * M:(b + 1) * M, :] = (offt + rank).astype(jnp.int32)
        base = base + jnp.sum(ohb, axis=0, keepdims=True)


def _gate(x_flat, Wg, bg):
    pos2d, off, hi = pl.pallas_call(
        _gate_body,
        out_shape=[
            jax.ShapeDtypeStruct((T, 1), jnp.int32),
            jax.ShapeDtypeStruct((1, EP), jnp.int32),
            jax.ShapeDtypeStruct((1, EP), jnp.int32),
        ],
    )(x_flat, Wg, bg.reshape(1, E))
    return pos2d.reshape(T), off, hi


def _sc_kernel(body):
    return functools.partial(
        pl.kernel, body,
        mesh=plsc.VectorSubcoreMesh(core_axis_name="c",
                                    subcore_axis_name="s"),
        out_type=jax.ShapeDtypeStruct((T, D), jnp.float32),
        scratch_types=[
            pltpu.VMEM((RPW,), jnp.int32),
            pltpu.VMEM((RPW, D), jnp.float32),
            pltpu.SemaphoreType.DMA,
        ],
    )()


def _dispatch(x_flat, pos):
    def body(x_hbm, pos_hbm, xs_hbm, idx_v, rows_v, sem):
        wid = lax.axis_index("s") * NC + lax.axis_index("c")
        base = wid * RPW
        pltpu.sync_copy(pos_hbm.at[pl.ds(base, RPW)], idx_v)
        pltpu.sync_copy(x_hbm.at[pl.ds(base, RPW)], rows_v)
        pltpu.async_copy(rows_v, xs_hbm.at[idx_v], sem).wait()

    return _sc_kernel(body)(x_flat, pos)


def _combine(ys, pos):
    def body(ys_hbm, pos_hbm, out_hbm, idx_v, rows_v, sem):
        wid = lax.axis_index("s") * NC + lax.axis_index("c")
        base = wid * RPW
        pltpu.sync_copy(pos_hbm.at[pl.ds(base, RPW)], idx_v)
        pltpu.async_copy(ys_hbm.at[idx_v], rows_v, sem).wait()
        pltpu.sync_copy(rows_v, out_hbm.at[pl.ds(base, RPW)])

    return _sc_kernel(body)(ys, pos)


def _ffn_body(off_ref, hi_ref, xs_ref, w1_ref, b1_ref, w2_ref, b2_ref,
              ys_ref):
    g = pl.program_id(0)

    @pl.when(g == 0)
    def _zero():
        ys_ref[...] = jnp.zeros_like(ys_ref)

    lo = off_ref[0, g]
    hi = hi_ref[0, g]
    t0 = lo // MF
    t1 = lax.select(hi > lo, (hi + MF - 1) // MF, t0)
    w1b = w1_ref[0].astype(jnp.bfloat16)
    w2b = w2_ref[0].astype(jnp.bfloat16)

    def tile_step(t, _):
        row = t * MF
        xt = xs_ref[pl.ds(row, MF), :].astype(jnp.bfloat16)
        h = jnp.maximum(
            jnp.dot(xt, w1b, preferred_element_type=jnp.float32)
            + b1_ref[0], 0.0)
        y = (jnp.dot(h.astype(jnp.bfloat16), w2b,
                     preferred_element_type=jnp.float32) + b2_ref[0])
        gidx = row + lax.broadcasted_iota(jnp.int32, (MF, 1), 0)
        msk = (gidx >= lo) & (gidx < hi)
        ys_ref[pl.ds(row, MF), :] += jnp.where(msk, y, 0.0)
        return 0

    lax.fori_loop(t0, t1, tile_step, 0)


def _ffn(off, hi, xs, W1, b1, W2, b2):
    grid_spec = pltpu.PrefetchScalarGridSpec(
        num_scalar_prefetch=2,
        grid=(E,),
        in_specs=[
            pl.BlockSpec((T, D), lambda g, o, h: (0, 0)),
            pl.BlockSpec((1, D, F), lambda g, o, h: (g, 0, 0)),
            pl.BlockSpec((1, 1, F), lambda g, o, h: (g, 0, 0)),
            pl.BlockSpec((1, F, D), lambda g, o, h: (g, 0, 0)),
            pl.BlockSpec((1, 1, D), lambda g, o, h: (g, 0, 0)),
        ],
        out_specs=pl.BlockSpec((T, D), lambda g, o, h: (0, 0)),
    )
    return pl.pallas_call(
        _ffn_body,
        grid_spec=grid_spec,
        out_shape=jax.ShapeDtypeStruct((T, D), jnp.float32),
        compiler_params=pltpu.CompilerParams(
            dimension_semantics=("arbitrary",)),
    )(off, hi, xs, W1, b1.reshape(E, 1, F), W2, b2.reshape(E, 1, D))


def kernel(x, Wg, bg, W1, b1, W2, b2):
    B, S, _ = x.shape
    x_flat = x.reshape(T, D)
    pos, off, hi = _gate(x_flat, Wg, bg)
    xs = _dispatch(x_flat, pos)
    ys = _ffn(off, hi, xs, W1, b1, W2, b2)
    out = _combine(ys, pos)
    return out.reshape(B, S, D)


# speedup vs baseline: 1.0475x; 1.0475x over previous
import functools

import jax
import jax.numpy as jnp
from jax import lax
from jax.experimental import pallas as pl
from jax.experimental.pallas import tpu as pltpu
from jax.experimental.pallas import tpu_sc as plsc

D = 768
E = 16
F = 2048
T = 2048
M = 128
MF = 256
NT = T // M
EP = 128

NC = 2
NS = 16
NW = NC * NS
RPW = T // NW


def _gate_body(x_ref, wg_ref, bg_ref, pos_ref, off_ref, hi_ref):
    xx = x_ref[...]
    logits16 = jnp.dot(xx, wg_ref[...],
                       preferred_element_type=jnp.float32) + bg_ref[...]
    logits = jnp.concatenate(
        [logits16, jnp.full((T, EP - E), -1e30, jnp.float32)], axis=1)
    lane = lax.broadcasted_iota(jnp.int32, (T, EP), 1)
    mx = jnp.max(logits, axis=-1, keepdims=True)
    e_t = jnp.min(jnp.where(logits == mx, lane, EP), axis=-1, keepdims=True)
    oh = (lane == e_t).astype(jnp.float32)

    counts = jnp.sum(oh, axis=0, keepdims=True)

    r = lax.broadcasted_iota(jnp.int32, (EP, EP), 0)
    c = lax.broadcasted_iota(jnp.int32, (EP, EP), 1)
    lt = (r < c).astype(jnp.float32)
    off = jnp.dot(counts, lt, preferred_element_type=jnp.float32)
    off_ref[...] = off.astype(jnp.int32)
    hi_ref[...] = (off + counts).astype(jnp.int32)

    ri = lax.broadcasted_iota(jnp.int32, (M, M), 0)
    ci = lax.broadcasted_iota(jnp.int32, (M, M), 1)
    tri = (ci <= ri).astype(jnp.float32)
    base = jnp.zeros((1, EP), jnp.float32)
    for b in range(NT):
        ohb = oh[b * M:(b + 1) * M]
        incb = jnp.dot(tri, ohb, preferred_element_type=jnp.float32) + base
        rank = jnp.sum(incb * ohb, axis=-1, keepdims=True) - 1.0
        offt = jnp.sum(off * ohb, axis=-1, keepdims=True)
        pos_ref[b * M:(b + 1) * M, :] = (offt + rank).astype(jnp.int32)
        base = base + jnp.sum(ohb, axis=0, keepdims=True)


def _gate(x_flat, Wg, bg):
    pos2d, off, hi = pl.pallas_call(
        _gate_body,
        out_shape=[
            jax.ShapeDtypeStruct((T, 1), jnp.int32),
            jax.ShapeDtypeStruct((1, EP), jnp.int32),
            jax.ShapeDtypeStruct((1, EP), jnp.int32),
        ],
    )(x_flat, Wg, bg.reshape(1, E))
    return pos2d.reshape(T), off, hi


def _sc_kernel(body):
    return functools.partial(
        pl.kernel, body,
        mesh=plsc.VectorSubcoreMesh(core_axis_name="c",
                                    subcore_axis_name="s"),
        out_type=jax.ShapeDtypeStruct((T, D), jnp.float32),
        scratch_types=[
            pltpu.VMEM((RPW,), jnp.int32),
            pltpu.VMEM((RPW, D), jnp.float32),
            pltpu.SemaphoreType.DMA,
        ],
    )()


def _dispatch(x_flat, pos):
    def body(x_hbm, pos_hbm, xs_hbm, idx_v, rows_v, sem):
        wid = lax.axis_index("s") * NC + lax.axis_index("c")
        base = wid * RPW
        pltpu.sync_copy(pos_hbm.at[pl.ds(base, RPW)], idx_v)
        pltpu.sync_copy(x_hbm.at[pl.ds(base, RPW)], rows_v)
        pltpu.async_copy(rows_v, xs_hbm.at[idx_v], sem).wait()

    return _sc_kernel(body)(x_flat, pos)


def _combine(ys, pos):
    def body(ys_hbm, pos_hbm, out_hbm, idx_v, rows_v, sem):
        wid = lax.axis_index("s") * NC + lax.axis_index("c")
        base = wid * RPW
        pltpu.sync_copy(pos_hbm.at[pl.ds(base, RPW)], idx_v)
        pltpu.async_copy(ys_hbm.at[idx_v], rows_v, sem).wait()
        pltpu.sync_copy(rows_v, out_hbm.at[pl.ds(base, RPW)])

    return _sc_kernel(body)(ys, pos)


def _ffn_body(off_ref, hi_ref, xs_ref, w1_ref, b1_ref, w2_ref, b2_ref,
              ys_ref):
    g = pl.program_id(0)

    @pl.when(g == 0)
    def _zero():
        ys_ref[...] = jnp.zeros_like(ys_ref)

    lo = off_ref[0, g]
    hi = hi_ref[0, g]
    t0 = lo // MF
    t1 = lax.select(hi > lo, (hi + MF - 1) // MF, t0)
    def tile_step(t, _):
        row = t * MF
        xt = xs_ref[pl.ds(row, MF), :]
        h = jnp.maximum(
            jnp.dot(xt, w1_ref[0], preferred_element_type=jnp.float32)
            + b1_ref[0], 0.0)
        y = (jnp.dot(h, w2_ref[0],
                     preferred_element_type=jnp.float32) + b2_ref[0])
        gidx = row + lax.broadcasted_iota(jnp.int32, (MF, 1), 0)
        msk = (gidx >= lo) & (gidx < hi)
        ys_ref[pl.ds(row, MF), :] += jnp.where(msk, y, 0.0)
        return 0

    lax.fori_loop(t0, t1, tile_step, 0)


def _ffn(off, hi, xs, W1, b1, W2, b2):
    grid_spec = pltpu.PrefetchScalarGridSpec(
        num_scalar_prefetch=2,
        grid=(E,),
        in_specs=[
            pl.BlockSpec((T, D), lambda g, o, h: (0, 0)),
            pl.BlockSpec((1, D, F), lambda g, o, h: (g, 0, 0)),
            pl.BlockSpec((1, 1, F), lambda g, o, h: (g, 0, 0)),
            pl.BlockSpec((1, F, D), lambda g, o, h: (g, 0, 0)),
            pl.BlockSpec((1, 1, D), lambda g, o, h: (g, 0, 0)),
        ],
        out_specs=pl.BlockSpec((T, D), lambda g, o, h: (0, 0)),
    )
    return pl.pallas_call(
        _ffn_body,
        grid_spec=grid_spec,
        out_shape=jax.ShapeDtypeStruct((T, D), jnp.float32),
        compiler_params=pltpu.CompilerParams(
            dimension_semantics=("arbitrary",)),
    )(off, hi, xs, W1, b1.reshape(E, 1, F), W2, b2.reshape(E, 1, D))


def kernel(x, Wg, bg, W1, b1, W2, b2):
    B, S, _ = x.shape
    x_flat = x.reshape(T, D)
    pos, off, hi = _gate(x_flat, Wg, bg)
    xs = _dispatch(x_flat, pos)
    ys = _ffn(off, hi, xs, W1, b1, W2, b2)
    out = _combine(ys, pos)
    return out.reshape(B, S, D)


# speedup vs baseline: 1.1124x; 1.0619x over previous
import functools

import jax
import jax.numpy as jnp
from jax import lax
from jax.experimental import pallas as pl
from jax.experimental.pallas import tpu as pltpu
from jax.experimental.pallas import tpu_sc as plsc

D = 768
E = 16
F = 2048
T = 2048
M = 128
MF = 128
NT = T // M
EP = 128

NC = 2
NS = 16
NW = NC * NS
RPW = T // NW


def _gate_body(x_ref, wg_ref, bg_ref, pos_ref, off_ref, hi_ref):
    xx = x_ref[...]
    logits16 = jnp.dot(xx, wg_ref[...],
                       preferred_element_type=jnp.float32) + bg_ref[...]
    logits = jnp.concatenate(
        [logits16, jnp.full((T, EP - E), -1e30, jnp.float32)], axis=1)
    lane = lax.broadcasted_iota(jnp.int32, (T, EP), 1)
    mx = jnp.max(logits, axis=-1, keepdims=True)
    e_t = jnp.min(jnp.where(logits == mx, lane, EP), axis=-1, keepdims=True)
    oh = (lane == e_t).astype(jnp.float32)

    counts = jnp.sum(oh, axis=0, keepdims=True)

    r = lax.broadcasted_iota(jnp.int32, (EP, EP), 0)
    c = lax.broadcasted_iota(jnp.int32, (EP, EP), 1)
    lt = (r < c).astype(jnp.float32)
    off = jnp.dot(counts, lt, preferred_element_type=jnp.float32)
    off_ref[...] = off.astype(jnp.int32)
    hi_ref[...] = (off + counts).astype(jnp.int32)

    ri = lax.broadcasted_iota(jnp.int32, (M, M), 0)
    ci = lax.broadcasted_iota(jnp.int32, (M, M), 1)
    tri = (ci <= ri).astype(jnp.float32)
    base = jnp.zeros((1, EP), jnp.float32)
    for b in range(NT):
        ohb = oh[b * M:(b + 1) * M]
        incb = jnp.dot(tri, ohb, preferred_element_type=jnp.float32) + base
        rank = jnp.sum(incb * ohb, axis=-1, keepdims=True) - 1.0
        offt = jnp.sum(off * ohb, axis=-1, keepdims=True)
        pos_ref[b * M:(b + 1) * M, :] = (offt + rank).astype(jnp.int32)
        base = base + jnp.sum(ohb, axis=0, keepdims=True)


def _gate(x_flat, Wg, bg):
    pos2d, off, hi = pl.pallas_call(
        _gate_body,
        out_shape=[
            jax.ShapeDtypeStruct((T, 1), jnp.int32),
            jax.ShapeDtypeStruct((1, EP), jnp.int32),
            jax.ShapeDtypeStruct((1, EP), jnp.int32),
        ],
    )(x_flat, Wg, bg.reshape(1, E))
    return pos2d.reshape(T), off, hi


def _sc_kernel(body):
    return functools.partial(
        pl.kernel, body,
        mesh=plsc.VectorSubcoreMesh(core_axis_name="c",
                                    subcore_axis_name="s"),
        out_type=jax.ShapeDtypeStruct((T, D), jnp.float32),
        scratch_types=[
            pltpu.VMEM((RPW,), jnp.int32),
            pltpu.VMEM((RPW, D), jnp.float32),
            pltpu.SemaphoreType.DMA,
        ],
    )()


def _dispatch(x_flat, pos):
    def body(x_hbm, pos_hbm, xs_hbm, idx_v, rows_v, sem):
        wid = lax.axis_index("s") * NC + lax.axis_index("c")
        base = wid * RPW
        pltpu.sync_copy(pos_hbm.at[pl.ds(base, RPW)], idx_v)
        pltpu.sync_copy(x_hbm.at[pl.ds(base, RPW)], rows_v)
        pltpu.async_copy(rows_v, xs_hbm.at[idx_v], sem).wait()

    return _sc_kernel(body)(x_flat, pos)


def _combine(ys, pos):
    def body(ys_hbm, pos_hbm, out_hbm, idx_v, rows_v, sem):
        wid = lax.axis_index("s") * NC + lax.axis_index("c")
        base = wid * RPW
        pltpu.sync_copy(pos_hbm.at[pl.ds(base, RPW)], idx_v)
        pltpu.async_copy(ys_hbm.at[idx_v], rows_v, sem).wait()
        pltpu.sync_copy(rows_v, out_hbm.at[pl.ds(base, RPW)])

    return _sc_kernel(body)(ys, pos)


def _ffn_body(off_ref, hi_ref, xs_ref, w1a_ref, w1b_ref, b1_ref, w2a_ref,
              w2b_ref, b2_ref, ys_ref):
    g = pl.program_id(0)

    @pl.when(g == 0)
    def _zero():
        ys_ref[...] = jnp.zeros_like(ys_ref)

    lo = off_ref[0, g]
    hi = hi_ref[0, g]
    t0 = lo // MF
    t1 = lax.select(hi > lo, (hi + MF - 1) // MF, t0)
    FH = F // 2

    def tile_step(t, _):
        row = t * MF
        xt = xs_ref[pl.ds(row, MF), :]
        ha = jnp.maximum(
            jnp.dot(xt, w1a_ref[0], preferred_element_type=jnp.float32)
            + b1_ref[0, :, :FH], 0.0)
        hb = jnp.maximum(
            jnp.dot(xt, w1b_ref[0], preferred_element_type=jnp.float32)
            + b1_ref[0, :, FH:], 0.0)
        y = (jnp.dot(ha, w2a_ref[0], preferred_element_type=jnp.float32)
             + jnp.dot(hb, w2b_ref[0], preferred_element_type=jnp.float32)
             + b2_ref[0])
        gidx = row + lax.broadcasted_iota(jnp.int32, (MF, 1), 0)
        msk = (gidx >= lo) & (gidx < hi)
        ys_ref[pl.ds(row, MF), :] += jnp.where(msk, y, 0.0)
        return 0

    lax.fori_loop(t0, t1, tile_step, 0)


def _ffn(off, hi, xs, W1, b1, W2, b2):
    grid_spec = pltpu.PrefetchScalarGridSpec(
        num_scalar_prefetch=2,
        grid=(E,),
        in_specs=[
            pl.BlockSpec((T, D), lambda g, o, h: (0, 0)),
            pl.BlockSpec((1, D, F // 2), lambda g, o, h: (g, 0, 0)),
            pl.BlockSpec((1, D, F // 2), lambda g, o, h: (g, 0, 1)),
            pl.BlockSpec((1, 1, F), lambda g, o, h: (g, 0, 0)),
            pl.BlockSpec((1, F // 2, D), lambda g, o, h: (g, 0, 0)),
            pl.BlockSpec((1, F // 2, D), lambda g, o, h: (g, 1, 0)),
            pl.BlockSpec((1, 1, D), lambda g, o, h: (g, 0, 0)),
        ],
        out_specs=pl.BlockSpec((T, D), lambda g, o, h: (0, 0)),
    )
    return pl.pallas_call(
        _ffn_body,
        grid_spec=grid_spec,
        out_shape=jax.ShapeDtypeStruct((T, D), jnp.float32),
        compiler_params=pltpu.CompilerParams(
            dimension_semantics=("arbitrary",)),
    )(off, hi, xs, W1, W1, b1.reshape(E, 1, F), W2, W2,
      b2.reshape(E, 1, D))


def kernel(x, Wg, bg, W1, b1, W2, b2):
    B, S, _ = x.shape
    x_flat = x.reshape(T, D)
    pos, off, hi = _gate(x_flat, Wg, bg)
    xs = _dispatch(x_flat, pos)
    ys = _ffn(off, hi, xs, W1, b1, W2, b2)
    out = _combine(ys, pos)
    return out.reshape(B, S, D)
